# baseline (speedup 1.0000x reference)
import functools

import jax
import jax.numpy as jnp
from jax import lax
from jax.experimental import pallas as pl
from jax.experimental.pallas import tpu as pltpu
from jax.experimental.pallas import tpu_sc as plsc

_LANES = 16
_Q = 4


@functools.lru_cache(maxsize=None)
def _build(B, S, E, V):
    info = plsc.get_sparse_core_info()
    nw = info.num_cores * info.num_subcores
    assert E % _LANES == 0
    bq = B // _Q
    n_units = S * _Q // nw
    assert S * _Q % nw == 0 and bq % 128 == 0
    su = nw // _Q
    e_vecs = E // _LANES
    bqp = bq + 8
    chunks = [(off, 128) for off in range(0, bq, 128)]

    mesh = plsc.VectorSubcoreMesh(core_axis_name="c", subcore_axis_name="s")

    @functools.partial(
        pl.kernel,
        mesh=mesh,
        out_type=jax.ShapeDtypeStruct((S, E, B), jnp.float32),
        scratch_types=[
            pltpu.VMEM((2, bq), jnp.int32),
            pltpu.VMEM((2, bq, E), jnp.float32),
            pltpu.VMEM((2, E, bqp), jnp.float32),
            pltpu.VMEM((S, E), jnp.float32),
            pltpu.SemaphoreType.DMA,
            pltpu.SemaphoreType.DMA,
            pltpu.SemaphoreType.DMA,
            pltpu.SemaphoreType.DMA,
            pltpu.SemaphoreType.DMA,
            pltpu.SemaphoreType.DMA,
        ],
        compiler_params=pltpu.CompilerParams(
            use_tc_tiling_on_sc=False, needs_layout_passes=False),
    )
    def k(xt_hbm, tok_hbm, pos_hbm, out_hbm, idx_v, g_v, t_v, pos_v,
          si0, si1, sg0, sg1, ss0, ss1):
        wid = lax.axis_index("s") * info.num_cores + lax.axis_index("c")
        q = lax.rem(wid, _Q)
        s_base = lax.div(wid, _Q)
        qb = q * bq
        sem_i = (si0, si1)
        sem_g = (sg0, sg1)
        sem_s = (ss0, ss1)

        pltpu.sync_copy(pos_hbm, pos_v)

        def s_of(t):
            return s_base + su * t

        def fetch_idx(t, u):
            pltpu.async_copy(
                xt_hbm.at[s_of(t)].at[pl.ds(qb, bq)], idx_v.at[u], sem_i[u])

        def wait_idx(u):
            pltpu.make_async_copy(
                xt_hbm.at[0].at[pl.ds(0, bq)], idx_v.at[u], sem_i[u]).wait()

        def fetch_g(u):
            for off, sz in chunks:
                pltpu.async_copy(
                    tok_hbm.at[idx_v.at[u].at[pl.ds(off, sz)]],
                    g_v.at[u].at[pl.ds(off, sz)],
                    sem_g[u])

        def wait_g(u):
            pltpu.make_async_copy(
                tok_hbm.at[pl.ds(0, bq)], g_v.at[u], sem_g[u]).wait()

        def store(t, u):
            s = s_of(t)
            pltpu.async_copy(
                t_v.at[u].at[:, pl.ds(0, bq)],
                out_hbm.at[s].at[:, pl.ds(qb, bq)],
                sem_s[u])

        def wait_s(u):
            pltpu.make_async_copy(
                out_hbm.at[0].at[:, pl.ds(0, bq)],
                t_v.at[u].at[:, pl.ds(0, bq)], sem_s[u]).wait()

        iota = lax.iota(jnp.int32, _LANES)

        def combine(t, u):
            s = s_of(t)
            pvs = [pos_v[s, pl.ds(j * _LANES, _LANES)] for j in range(e_vecs)]
            ejs = [j * _LANES + iota for j in range(e_vecs)]

            def body(r, carry):
                pv = carry
                for j in range(e_vecs):
                    v = g_v[u, r, pl.ds(j * _LANES, _LANES)] + pv[j]
                    plsc.store_scatter(
                        t_v.at[u], [ejs[j], jnp.broadcast_to(r, (_LANES,))], v)
                return pv
            lax.fori_loop(0, bq, body, tuple(pvs))

        def unit(t, u, pre_g, pre_i, w_s):
            if pre_g:
                wait_idx(1 - u)
                fetch_g(1 - u)
            wait_g(u)
            if pre_i:
                fetch_idx(t + 2, u)
            if w_s:
                wait_s(u)
            combine(t, u)
            store(t, u)

        assert n_units >= 5 and n_units % 2 == 1
        fetch_idx(0, 0)
        wait_idx(0)
        fetch_g(0)
        fetch_idx(1, 1)

        unit(0, 0, True, True, False)
        unit(1, 1, True, True, False)

        def group(g2, _):
            for uu in (0, 1):
                unit(2 + 2 * g2 + uu, uu, True, True, True)
            return 0

        lax.fori_loop(0, (n_units - 5) // 2, group, 0)

        unit(n_units - 3, 0, True, True, True)
        unit(n_units - 2, 1, True, False, True)
        unit(n_units - 1, 0, False, False, True)
        wait_s(1)
        wait_s(0)

    return k


def kernel(x, token_table, pos_table):
    B, S = x.shape
    V, E = token_table.shape
    k = _build(B, S, E, V)
    xt = x.astype(jnp.int32).T
    out_t = k(xt, token_table, pos_table)
    return out_t.transpose(2, 0, 1)


# speedup vs baseline: 1.2796x; 1.2796x over previous
import functools

import jax
import jax.numpy as jnp
from jax import lax
from jax.experimental import pallas as pl
from jax.experimental.pallas import tpu as pltpu
from jax.experimental.pallas import tpu_sc as plsc

_LANES = 16


@functools.lru_cache(maxsize=None)
def _build(B, S, E, V):
    info = plsc.get_sparse_core_info()
    nw = info.num_cores * info.num_subcores
    assert B % nw == 0, (B, nw)
    assert E % _LANES == 0 and S % 2 == 0
    rpw = B // nw
    assert rpw >= 6 and rpw % 2 == 0
    e_vecs = E // _LANES
    s2 = S // 2
    wide = 2 * E
    chunks = []
    off = 0
    while off < S:
        sz = min(128, S - off)
        chunks.append((off, sz))
        off += sz

    mesh = plsc.VectorSubcoreMesh(core_axis_name="c", subcore_axis_name="s")

    @functools.partial(
        pl.kernel,
        mesh=mesh,
        out_type=jax.ShapeDtypeStruct((B, s2, wide), jnp.float32),
        scratch_types=[
            pltpu.VMEM((rpw * S,), jnp.int32),
            pltpu.VMEM((3, S, E), jnp.float32),
            pltpu.VMEM((3, s2, wide), jnp.float32),
            pltpu.VMEM((s2, wide), jnp.float32),
            pltpu.SemaphoreType.DMA,
            pltpu.SemaphoreType.DMA,
            pltpu.SemaphoreType.DMA,
            pltpu.SemaphoreType.DMA,
            pltpu.SemaphoreType.DMA,
            pltpu.SemaphoreType.DMA,
        ],
        compiler_params=pltpu.CompilerParams(use_tc_tiling_on_sc=False),
    )
    def k(x_hbm, tok_hbm, pos_hbm, out_hbm, idx_v, g_v, rows_v, pos_v,
          sg0, sg1, sg2, ss0, ss1, ss2):
        wid = lax.axis_index("s") * info.num_cores + lax.axis_index("c")
        base = wid * rpw
        sem_g = (sg0, sg1, sg2)
        sem_s = (ss0, ss1, ss2)

        pltpu.sync_copy(x_hbm.at[pl.ds(base * S, rpw * S)], idx_v)
        pltpu.sync_copy(pos_hbm, pos_v)

        def fetch(i, u):
            for off, sz in chunks:
                pltpu.async_copy(
                    tok_hbm.at[idx_v.at[pl.ds(i * S + off, sz)]],
                    g_v.at[u].at[pl.ds(off, sz)],
                    sem_g[u])

        def wait_g(u):
            pltpu.make_async_copy(
                tok_hbm.at[pl.ds(0, S)], g_v.at[u], sem_g[u]).wait()

        def store(i, u):
            pltpu.async_copy(rows_v.at[u], out_hbm.at[base + i], sem_s[u])

        def wait_s(u):
            pltpu.make_async_copy(out_hbm.at[0], rows_v.at[u], sem_s[u]).wait()

        def add_pos(u):
            def body(p, _):
                for h in (0, 1):
                    for j in range(e_vecs):
                        src = pl.ds(j * _LANES, _LANES)
                        dst = pl.ds(h * E + j * _LANES, _LANES)
                        rows_v[u, p, dst] = g_v[u, 2 * p + h, src] + pos_v[p, dst]
                return 0
            lax.fori_loop(0, s2, body, 0)

        assert rpw % 3 == 2 and rpw >= 8
        fetch(0, 0)
        fetch(1, 1)

        def iteration(i, u, pre, w_s):
            wait_g(u)
            if pre:
                fetch(i + 2, (u + 2) % 3)
            if w_s:
                wait_s(u)
            add_pos(u)
            store(i, u)

        iteration(0, 0, True, False)
        iteration(1, 1, True, False)
        iteration(2, 2, True, False)

        def group(g, _):
            for uu in (0, 1, 2):
                iteration(3 + 3 * g + uu, uu, True, True)
            return 0

        lax.fori_loop(0, (rpw - 5) // 3, group, 0)

        iteration(rpw - 2, (rpw - 2) % 3, False, True)
        iteration(rpw - 1, (rpw - 1) % 3, False, True)

        wait_s((rpw - 3) % 3)
        wait_s((rpw - 2) % 3)
        wait_s((rpw - 1) % 3)

    return k


def kernel(x, token_table, pos_table):
    B, S = x.shape
    V, E = token_table.shape
    k = _build(B, S, E, V)
    pos2 = pos_table.reshape(S // 2, 2 * E)
    x1 = x.astype(jnp.int32).reshape(B * S)
    out = k(x1, token_table, pos2)
    return out.reshape(B, S, E)
